# baseline (speedup 1.0000x reference)
import functools

import jax
import jax.numpy as jnp
from jax import lax
from jax.experimental import pallas as pl
from jax.experimental.pallas import tpu as pltpu
from jax.experimental.pallas import tpu_sc as plsc

BATCH = 4096
HIST = 50
EMBED_DIM = 64

NUM_CORES = 2
NUM_SUBCORES = 16
NUM_WORKERS = NUM_CORES * NUM_SUBCORES
BATCH_PER_WORKER = BATCH // NUM_WORKERS
GROUP_B = 16
N_GROUPS = BATCH_PER_WORKER // GROUP_B
GROUP_IDX = GROUP_B * HIST
HCH = 5
N_HCH = HIST // HCH
BPAD = GROUP_B + 1


def _build():
    mesh = plsc.VectorSubcoreMesh(core_axis_name="c", subcore_axis_name="s")

    @functools.partial(
        pl.kernel,
        mesh=mesh,
        out_type=jax.ShapeDtypeStruct((HIST, 8, NUM_WORKERS, 8, 128),
                                      jnp.float32),
        scratch_types=[
            pltpu.VMEM((N_GROUPS, GROUP_IDX), jnp.int32),
            pltpu.VMEM((2, GROUP_IDX, EMBED_DIM), jnp.float32),
            pltpu.VMEM((2, HCH, 8, 8, BPAD), jnp.float32),
            pltpu.SemaphoreType.DMA((2,)),
            pltpu.SemaphoreType.DMA((2,)),
        ],
        compiler_params=pltpu.CompilerParams(use_tc_tiling_on_sc=False,
                                             needs_layout_passes=False),
    )
    def gather_kernel(idx_hbm, table_hbm, out_hbm, idx_v, rows_v, tbuf, gsem,
                      wsem):
        wid = lax.axis_index("s") * NUM_CORES + lax.axis_index("c")

        pltpu.sync_copy(idx_hbm.at[wid], idx_v)

        iota = lax.iota(jnp.int32, 16)
        zero16 = iota * 0
        tq = [lax.shift_right_logical(iota + 16 * q, 3) for q in range(4)]
        rq = [lax.bitwise_and(iota + 16 * q, 7) for q in range(4)]

        pltpu.async_copy(table_hbm.at[idx_v.at[0]], rows_v.at[0], gsem.at[0])

        def wait_write(par):
            pltpu.make_async_copy(
                tbuf.at[par, pl.ds(0, HCH), :, :, pl.ds(0, GROUP_B)],
                out_hbm.at[pl.ds(0, HCH), pl.ds(0, 8), wid, pl.ds(0, 8),
                           pl.ds(0, GROUP_B)],
                wsem.at[par]).wait()

        def do_chunk(rows, g, hc, par):
            def hist_body(hh, _):
                h = hc * HCH + hh
                hsp = zero16 + hh

                def batch_body(bl, _):
                    bsp = zero16 + bl
                    row = bl * HIST + h
                    for q in range(4):
                        v = rows[row, pl.ds(16 * q, 16)]
                        plsc.store_scatter(tbuf.at[par],
                                           [hsp, tq[q], rq[q], bsp], v)
                    return ()

                lax.fori_loop(0, GROUP_B, batch_body, (), unroll=False)
                return ()

            lax.fori_loop(0, HCH, hist_body, (), unroll=False)
            pltpu.async_copy(
                tbuf.at[par, pl.ds(0, HCH), :, :, pl.ds(0, GROUP_B)],
                out_hbm.at[pl.ds(hc * HCH, HCH), pl.ds(0, 8), wid,
                           pl.ds(0, 8), pl.ds(g * GROUP_B, GROUP_B)],
                wsem.at[par])

        for g in range(N_GROUPS):
            b = g % 2
            pltpu.make_async_copy(table_hbm.at[idx_v.at[g]], rows_v.at[b],
                                  gsem.at[b]).wait()
            if g + 1 < N_GROUPS:
                pltpu.async_copy(table_hbm.at[idx_v.at[g + 1]],
                                 rows_v.at[1 - b], gsem.at[1 - b])
            rows = rows_v.at[b]

            if g == 0:
                do_chunk(rows, 0, 0, 0)
                do_chunk(rows, 0, 1, 1)
                start = 2
            else:
                start = 0

            def hc_body(hc, _, g=g, rows=rows):
                par = lax.rem(hc, 2)
                for p in range(2):

                    def run(p=p):
                        wait_write(p)
                        do_chunk(rows, g, hc, p)

                    pl.when(par == p)(run)
                return ()

            lax.fori_loop(start, N_HCH, hc_body, (), unroll=False)

        for par in range(2):
            wait_write(par)

    return gather_kernel


_gather = _build()


@jax.jit
def kernel(x, table):
    idx3d = x.reshape(NUM_WORKERS, N_GROUPS, GROUP_IDX)
    out5d = _gather(idx3d, table)
    return out5d.transpose(2, 4, 0, 1, 3).reshape(BATCH, HIST, EMBED_DIM)


# speedup vs baseline: 1.0136x; 1.0136x over previous
import functools

import jax
import jax.numpy as jnp
from jax import lax
from jax.experimental import pallas as pl
from jax.experimental.pallas import tpu as pltpu
from jax.experimental.pallas import tpu_sc as plsc

BATCH = 4096
HIST = 50
EMBED_DIM = 64

NUM_CORES = 2
NUM_SUBCORES = 16
NUM_WORKERS = NUM_CORES * NUM_SUBCORES
BATCH_PER_WORKER = BATCH // NUM_WORKERS
GROUP_B = 16
N_GROUPS = BATCH_PER_WORKER // GROUP_B
GROUP_IDX = GROUP_B * HIST
HCH = 5
N_HCH = HIST // HCH
BPAD = GROUP_B + 1


def _build():
    mesh = plsc.VectorSubcoreMesh(core_axis_name="c", subcore_axis_name="s")

    @functools.partial(
        pl.kernel,
        mesh=mesh,
        out_type=jax.ShapeDtypeStruct((HIST, 8, NUM_WORKERS, 8, 128),
                                      jnp.float32),
        scratch_types=[
            pltpu.VMEM((N_GROUPS, GROUP_IDX), jnp.int32),
            pltpu.VMEM((2, GROUP_IDX, EMBED_DIM), jnp.float32),
            pltpu.VMEM((2, HCH, 8, 8, BPAD), jnp.float32),
            pltpu.SemaphoreType.DMA((2,)),
            pltpu.SemaphoreType.DMA((2,)),
        ],
        compiler_params=pltpu.CompilerParams(use_tc_tiling_on_sc=False,
                                             needs_layout_passes=False),
    )
    def gather_kernel(idx_hbm, table_hbm, out_hbm, idx_v, rows_v, tbuf, gsem,
                      wsem):
        wid = lax.axis_index("s") * NUM_CORES + lax.axis_index("c")

        pltpu.sync_copy(idx_hbm.at[wid], idx_v)

        iota = lax.iota(jnp.int32, 16)
        zero16 = iota * 0
        tq = [lax.shift_right_logical(iota + 16 * q, 3) for q in range(4)]
        rq = [lax.bitwise_and(iota + 16 * q, 7) for q in range(4)]

        pltpu.async_copy(table_hbm.at[idx_v.at[0]], rows_v.at[0], gsem.at[0])

        def wait_write(par):
            pltpu.make_async_copy(
                tbuf.at[par, pl.ds(0, HCH), :, :, pl.ds(0, GROUP_B)],
                out_hbm.at[pl.ds(0, HCH), pl.ds(0, 8), wid, pl.ds(0, 8),
                           pl.ds(0, GROUP_B)],
                wsem.at[par]).wait()

        def do_chunk(rows, g, hc, par):
            def hist_body(hh, _):
                h = hc * HCH + hh
                hsp = zero16 + hh

                def batch_body(bl, _):
                    bsp = zero16 + bl
                    row = bl * HIST + h
                    for q in range(4):
                        v = rows[row, pl.ds(16 * q, 16)]
                        plsc.store_scatter(tbuf.at[par],
                                           [hsp, tq[q], rq[q], bsp], v)
                    return ()

                lax.fori_loop(0, GROUP_B, batch_body, (), unroll=4)
                return ()

            lax.fori_loop(0, HCH, hist_body, (), unroll=False)
            pltpu.async_copy(
                tbuf.at[par, pl.ds(0, HCH), :, :, pl.ds(0, GROUP_B)],
                out_hbm.at[pl.ds(hc * HCH, HCH), pl.ds(0, 8), wid,
                           pl.ds(0, 8), pl.ds(g * GROUP_B, GROUP_B)],
                wsem.at[par])

        for g in range(N_GROUPS):
            b = g % 2
            pltpu.make_async_copy(table_hbm.at[idx_v.at[g]], rows_v.at[b],
                                  gsem.at[b]).wait()
            if g + 1 < N_GROUPS:
                pltpu.async_copy(table_hbm.at[idx_v.at[g + 1]],
                                 rows_v.at[1 - b], gsem.at[1 - b])
            rows = rows_v.at[b]

            if g == 0:
                do_chunk(rows, 0, 0, 0)
                do_chunk(rows, 0, 1, 1)
                start = 2
            else:
                start = 0

            def hc_body(hc, _, g=g, rows=rows):
                par = lax.rem(hc, 2)
                for p in range(2):

                    def run(p=p):
                        wait_write(p)
                        do_chunk(rows, g, hc, p)

                    pl.when(par == p)(run)
                return ()

            lax.fori_loop(start, N_HCH, hc_body, (), unroll=False)

        for par in range(2):
            wait_write(par)

    return gather_kernel


_gather = _build()


@jax.jit
def kernel(x, table):
    idx3d = x.reshape(NUM_WORKERS, N_GROUPS, GROUP_IDX)
    out5d = _gather(idx3d, table)
    return out5d.transpose(2, 4, 0, 1, 3).reshape(BATCH, HIST, EMBED_DIM)
